# baseline (speedup 1.0000x reference)
import jax
import jax.numpy as jnp
from jax.experimental import pallas as pl

_RADIUS = 0.25
_NSAMPLE = 32
_QBLK = 1024
_JBLK = 256


def _smooth_body(pcq_ref, flq_ref, pca_ref, fla_ref, out_ref):
    nchunks = pca_ref.shape[1]
    jblk = pca_ref.shape[3]
    qblk = pcq_ref.shape[1]

    pos_q = pcq_ref[0]
    fl_q = flq_ref[0]
    r2 = jnp.float32(_RADIUS * _RADIUS)
    ns = jnp.float32(_NSAMPLE)
    pos_q2 = pos_q + pos_q
    fl_q2 = fl_q + fl_q
    psq_q = jnp.sum(pos_q * pos_q, axis=1, keepdims=True) - r2
    fsq_q = jnp.sum(fl_q * fl_q, axis=1, keepdims=True)

    rowi = jax.lax.broadcasted_iota(jnp.int32, (jblk, jblk), 0)
    coli = jax.lax.broadcasted_iota(jnp.int32, (jblk, jblk), 1)
    tri = (rowi <= coli).astype(jnp.float32)

    def chunk(carry):
        j, cnt, acc, first = carry
        pos_j = pca_ref[0, j]
        fl_j = fla_ref[0, j]
        psq_j = jnp.sum(pos_j * pos_j, axis=0, keepdims=True)
        fsq_j = jnp.sum(fl_j * fl_j, axis=0, keepdims=True)
        inner_p2 = jax.lax.dot_general(
            pos_q2, pos_j, (((1,), (0,)), ((), ())),
            preferred_element_type=jnp.float32)
        inner_f2 = jax.lax.dot_general(
            fl_q2, fl_j, (((1,), (0,)), ((), ())),
            preferred_element_type=jnp.float32)
        wf = ((psq_q + psq_j) < inner_p2).astype(jnp.float32)
        s = (fsq_q + fsq_j) - inner_f2
        nrm_w = wf * jnp.sqrt(jnp.maximum(s, 0.0))
        ccount = jax.lax.dot_general(
            wf, tri, (((1,), (0,)), ((), ())),
            preferred_element_type=jnp.float32)
        rank = cnt + ccount
        acc = acc + jnp.sum(nrm_w * (rank <= ns).astype(jnp.float32),
                            axis=1, keepdims=True)
        first = first + jnp.sum(nrm_w * (rank == 1.0).astype(jnp.float32),
                                axis=1, keepdims=True)
        cnt = cnt + jnp.sum(wf, axis=1, keepdims=True)
        return j + 1, cnt, acc, first

    def cond(carry):
        j, cnt, _, _ = carry
        return (j < nchunks) & (jnp.min(cnt) < ns)

    init = (jnp.int32(0),
            jnp.zeros((qblk, 1), jnp.float32),
            jnp.zeros((qblk, 1), jnp.float32),
            jnp.zeros((qblk, 1), jnp.float32))
    _, cnt, acc, first = jax.lax.while_loop(cond, chunk, init)
    pad = jnp.maximum(ns - cnt, 0.0) * first
    out_ref[0] = jnp.sum(acc + pad, keepdims=True)


def kernel(flow, pc1):
    B, C, N = flow.shape
    nq = N // _QBLK
    nchunks = N // _JBLK

    pos_t = jnp.transpose(pc1, (0, 2, 1))
    flw_t = jnp.transpose(flow, (0, 2, 1))
    pos_ch = jnp.transpose(pc1.reshape(B, C, nchunks, _JBLK), (0, 2, 1, 3))
    flw_ch = jnp.transpose(flow.reshape(B, C, nchunks, _JBLK), (0, 2, 1, 3))

    partial = pl.pallas_call(
        _smooth_body,
        grid=(B, nq),
        in_specs=[
            pl.BlockSpec((1, _QBLK, C), lambda b, q: (b, q, 0)),
            pl.BlockSpec((1, _QBLK, C), lambda b, q: (b, q, 0)),
            pl.BlockSpec((1, nchunks, C, _JBLK), lambda b, q: (b, 0, 0, 0)),
            pl.BlockSpec((1, nchunks, C, _JBLK), lambda b, q: (b, 0, 0, 0)),
        ],
        out_specs=pl.BlockSpec((1, 1, 1), lambda b, q: (b * nq + q, 0, 0)),
        out_shape=jax.ShapeDtypeStruct((B * nq, 1, 1), jnp.float32),
    )(pos_t, flw_t, pos_ch, flw_ch)

    return jnp.sum(partial) / jnp.float32(B * _NSAMPLE)


# speedup vs baseline: 1.0347x; 1.0347x over previous
import jax
import jax.numpy as jnp
from jax.experimental import pallas as pl

_RADIUS = 0.25
_NSAMPLE = 32
_QBLK = 1024
_JBLK = 512


def _smooth_body(pcq_ref, flq_ref, pca_ref, fla_ref, out_ref):
    nchunks = pca_ref.shape[1]
    jblk = pca_ref.shape[3]
    qblk = pcq_ref.shape[1]

    pos_q = pcq_ref[0]
    fl_q = flq_ref[0]
    r2 = jnp.float32(_RADIUS * _RADIUS)
    ns = jnp.float32(_NSAMPLE)
    pos_q2 = pos_q + pos_q
    fl_q2 = fl_q + fl_q
    psq_q = jnp.sum(pos_q * pos_q, axis=1, keepdims=True) - r2
    fsq_q = jnp.sum(fl_q * fl_q, axis=1, keepdims=True)

    rowi = jax.lax.broadcasted_iota(jnp.int32, (jblk, jblk), 0)
    coli = jax.lax.broadcasted_iota(jnp.int32, (jblk, jblk), 1)
    tri = (rowi <= coli).astype(jnp.float32)

    def chunk(carry):
        j, cnt, acc, first = carry
        pos_j = pca_ref[0, j]
        fl_j = fla_ref[0, j]
        psq_j = jnp.sum(pos_j * pos_j, axis=0, keepdims=True)
        fsq_j = jnp.sum(fl_j * fl_j, axis=0, keepdims=True)
        inner_p2 = jax.lax.dot_general(
            pos_q2, pos_j, (((1,), (0,)), ((), ())),
            preferred_element_type=jnp.float32)
        inner_f2 = jax.lax.dot_general(
            fl_q2, fl_j, (((1,), (0,)), ((), ())),
            preferred_element_type=jnp.float32)
        wf = ((psq_q + psq_j) < inner_p2).astype(jnp.float32)
        s = (fsq_q + fsq_j) - inner_f2
        nrm_w = jnp.sqrt(jnp.maximum(s * wf, 0.0))
        ccount = jax.lax.dot_general(
            wf, tri, (((1,), (0,)), ((), ())),
            preferred_element_type=jnp.float32)
        rank = cnt + ccount
        acc = acc + jnp.sum(nrm_w * (rank <= ns).astype(jnp.float32),
                            axis=1, keepdims=True)
        first = first + jnp.sum(nrm_w * (rank == 1.0).astype(jnp.float32),
                                axis=1, keepdims=True)
        cnt = cnt + jnp.sum(wf, axis=1, keepdims=True)
        return j + 1, cnt, acc, first

    def cond(carry):
        j, cnt, _, _ = carry
        return (j < nchunks) & (jnp.min(cnt) < ns)

    init = (jnp.int32(0),
            jnp.zeros((qblk, 1), jnp.float32),
            jnp.zeros((qblk, 1), jnp.float32),
            jnp.zeros((qblk, 1), jnp.float32))
    _, cnt, acc, first = jax.lax.while_loop(cond, chunk, init)
    pad = jnp.maximum(ns - cnt, 0.0) * first
    out_ref[0] = jnp.sum(acc + pad, keepdims=True)


def kernel(flow, pc1):
    B, C, N = flow.shape
    nq = N // _QBLK
    nchunks = N // _JBLK

    pos_t = jnp.transpose(pc1, (0, 2, 1))
    flw_t = jnp.transpose(flow, (0, 2, 1))
    pos_ch = jnp.transpose(pc1.reshape(B, C, nchunks, _JBLK), (0, 2, 1, 3))
    flw_ch = jnp.transpose(flow.reshape(B, C, nchunks, _JBLK), (0, 2, 1, 3))

    partial = pl.pallas_call(
        _smooth_body,
        grid=(B, nq),
        in_specs=[
            pl.BlockSpec((1, _QBLK, C), lambda b, q: (b, q, 0)),
            pl.BlockSpec((1, _QBLK, C), lambda b, q: (b, q, 0)),
            pl.BlockSpec((1, nchunks, C, _JBLK), lambda b, q: (b, 0, 0, 0)),
            pl.BlockSpec((1, nchunks, C, _JBLK), lambda b, q: (b, 0, 0, 0)),
        ],
        out_specs=pl.BlockSpec((1, 1, 1), lambda b, q: (b * nq + q, 0, 0)),
        out_shape=jax.ShapeDtypeStruct((B * nq, 1, 1), jnp.float32),
    )(pos_t, flw_t, pos_ch, flw_ch)

    return jnp.sum(partial) / jnp.float32(B * _NSAMPLE)


# speedup vs baseline: 1.1883x; 1.1485x over previous
import jax
import jax.numpy as jnp
from jax.experimental import pallas as pl

_RADIUS = 0.25
_NSAMPLE = 32
_QBLK = 1024
_JBLK = 512


def _smooth_body(pcq_ref, flq_ref, pca_ref, fla_ref, out_ref):
    nchunks = pca_ref.shape[1]
    jblk = pca_ref.shape[2]
    qblk = pcq_ref.shape[2]

    pos_q = pcq_ref[0]
    fl_q = flq_ref[0]
    r2 = jnp.float32(_RADIUS * _RADIUS)
    ns = jnp.float32(_NSAMPLE)
    pos_q2 = pos_q + pos_q
    fl_q2 = fl_q + fl_q
    psq_q = jnp.sum(pos_q * pos_q, axis=0, keepdims=True) - r2
    fsq_q = jnp.sum(fl_q * fl_q, axis=0, keepdims=True)

    rowi = jax.lax.broadcasted_iota(jnp.int32, (jblk, jblk), 0)
    coli = jax.lax.broadcasted_iota(jnp.int32, (jblk, jblk), 1)
    tri = (coli <= rowi).astype(jnp.float32)

    def chunk(carry):
        j, cnt, acc, first = carry
        pos_j = pca_ref[0, j]
        fl_j = fla_ref[0, j]
        psq_j = jnp.sum(pos_j * pos_j, axis=1, keepdims=True)
        fsq_j = jnp.sum(fl_j * fl_j, axis=1, keepdims=True)
        inner_p2 = jax.lax.dot_general(
            pos_j, pos_q2, (((1,), (0,)), ((), ())),
            preferred_element_type=jnp.float32)
        inner_f2 = jax.lax.dot_general(
            fl_j, fl_q2, (((1,), (0,)), ((), ())),
            preferred_element_type=jnp.float32)
        wf = ((psq_q + psq_j) < inner_p2).astype(jnp.float32)
        s = (fsq_q + fsq_j) - inner_f2
        nrm_w = wf * jnp.sqrt(jnp.maximum(s, 0.0))
        ccount = jax.lax.dot_general(
            tri, wf, (((1,), (0,)), ((), ())),
            preferred_element_type=jnp.float32)
        rank = cnt + ccount
        acc = acc + jnp.sum(nrm_w * (rank <= ns).astype(jnp.float32),
                            axis=0, keepdims=True)
        first = first + jnp.sum(nrm_w * (rank == 1.0).astype(jnp.float32),
                                axis=0, keepdims=True)
        cnt = cnt + jnp.sum(wf, axis=0, keepdims=True)
        return j + 1, cnt, acc, first

    def cond(carry):
        j, cnt, _, _ = carry
        return (j < nchunks) & (jnp.min(cnt) < ns)

    init = (jnp.int32(0),
            jnp.zeros((1, qblk), jnp.float32),
            jnp.zeros((1, qblk), jnp.float32),
            jnp.zeros((1, qblk), jnp.float32))
    _, cnt, acc, first = jax.lax.while_loop(cond, chunk, init)
    pad = jnp.maximum(ns - cnt, 0.0) * first
    out_ref[0] = jnp.sum(acc + pad, keepdims=True)


def kernel(flow, pc1):
    B, C, N = flow.shape
    nq = N // _QBLK
    nchunks = N // _JBLK

    pos_ch = jnp.transpose(pc1.reshape(B, C, nchunks, _JBLK), (0, 2, 3, 1))
    flw_ch = jnp.transpose(flow.reshape(B, C, nchunks, _JBLK), (0, 2, 3, 1))

    partial = pl.pallas_call(
        _smooth_body,
        grid=(B, nq),
        in_specs=[
            pl.BlockSpec((1, C, _QBLK), lambda b, q: (b, 0, q)),
            pl.BlockSpec((1, C, _QBLK), lambda b, q: (b, 0, q)),
            pl.BlockSpec((1, nchunks, _JBLK, C), lambda b, q: (b, 0, 0, 0)),
            pl.BlockSpec((1, nchunks, _JBLK, C), lambda b, q: (b, 0, 0, 0)),
        ],
        out_specs=pl.BlockSpec((1, 1, 1), lambda b, q: (b * nq + q, 0, 0)),
        out_shape=jax.ShapeDtypeStruct((B * nq, 1, 1), jnp.float32),
    )(pc1, flow, pos_ch, flw_ch)

    return jnp.sum(partial) / jnp.float32(B * _NSAMPLE)
